# baseline (speedup 1.0000x reference)
import jax
import jax.numpy as jnp
from jax import lax
from jax.experimental import pallas as pl
from jax.experimental.pallas import tpu as pltpu
from jax.experimental.pallas import tpu_sc as plsc

_H = 128
_B = 80
_G = _B // 16
_UNROLL = 8


def _prep_body(emb_ref, d_ref, scale_ref, a_ref, b_ref):
    x = emb_ref[...]
    ss = jnp.sum(x * x, axis=1, keepdims=True)
    norm = jnp.maximum(jnp.sqrt(ss), 1e-12)
    e = x / norm
    b_ref[...] = e
    a_ref[...] = e * (d_ref[...] * scale_ref[0])[None, :]


def _prep(emb, d, scale):
    return pl.pallas_call(
        _prep_body,
        out_shape=(
            jax.ShapeDtypeStruct(emb.shape, jnp.float32),
            jax.ShapeDtypeStruct(emb.shape, jnp.float32),
        ),
    )(emb, d, scale)


def _edge_body(a_hbm, b_hbm, src_hbm, dst_hbm, out_hbm,
               sidx, didx, outv, stage, ar0, br0, ar1, br1,
               sa0, sb0, sa1, sb1):
    ep = out_hbm.shape[0] // 32
    nb = ep // _B
    wid = lax.axis_index("s") * 2 + lax.axis_index("c")
    base = pl.multiple_of(wid * ep, 8)
    lane = lax.iota(jnp.int32, 16)

    pltpu.sync_copy(src_hbm.at[pl.ds(base, ep)], sidx)
    pltpu.sync_copy(dst_hbm.at[pl.ds(base, ep)], didx)

    bufs = ((ar0, br0, sa0, sb0), (ar1, br1, sa1, sb1))

    def start(ib, buf):
        ar, br, sa, sb = buf
        off = pl.multiple_of(ib * _B, 8)
        pltpu.async_copy(a_hbm.at[sidx.at[pl.ds(off, _B)]], ar, sa)
        pltpu.async_copy(b_hbm.at[didx.at[pl.ds(off, _B)]], br, sb)

    def wait(buf):
        ar, br, sa, sb = buf
        pltpu.make_async_copy(a_hbm.at[sidx.at[pl.ds(0, _B)]], ar, sa).wait()
        pltpu.make_async_copy(b_hbm.at[didx.at[pl.ds(0, _B)]], br, sb).wait()

    def tree_sum(vs):
        while len(vs) > 1:
            vs = [a + b for a, b in zip(vs[::2], vs[1::2])] + \
                 ([vs[-1]] if len(vs) % 2 else [])
        return vs[0]

    def compute(ib, buf):
        ar, br = buf[0], buf[1]

        def group_body(g, _):
            e0 = g * 16
            for j in range(16):
                e = e0 + j
                prods = []
                for c in range(_H // 16):
                    va = ar[e, pl.ds(c * 16, 16)]
                    vb = br[e, pl.ds(c * 16, 16)]
                    prods.append(va * vb)
                acc = tree_sum(prods)
                plsc.store_scatter(stage, [lane * 16 + j], acc)
            tot = tree_sum([stage[pl.ds(l * 16, 16)] for l in range(16)])
            o = pl.multiple_of(ib * _B + e0, 16)
            outv[pl.ds(o, 16)] = tot
            return 0

        lax.fori_loop(0, _G, group_body, 0)

    start(0, bufs[0])

    def pair_body(i2, _):
        ib = i2 * 2
        start(ib + 1, bufs[1])
        wait(bufs[0])
        compute(ib, bufs[0])
        start(ib + 2, bufs[0])
        wait(bufs[1])
        compute(ib + 1, bufs[1])
        return 0

    lax.fori_loop(0, (nb - 1) // 2, pair_body, 0)
    wait(bufs[0])
    compute(nb - 1, bufs[0])

    pltpu.sync_copy(outv, out_hbm.at[pl.ds(base, ep)])


def _edge_dot(a, b, src, dst):
    n_edges = src.shape[0]
    ep = n_edges // 32
    mesh = plsc.VectorSubcoreMesh(core_axis_name="c", subcore_axis_name="s")
    return pl.kernel(
        _edge_body,
        out_type=jax.ShapeDtypeStruct((n_edges,), jnp.float32),
        mesh=mesh,
        compiler_params=pltpu.CompilerParams(needs_layout_passes=False),
        scratch_types=[
            pltpu.VMEM((ep,), jnp.int32),
            pltpu.VMEM((ep,), jnp.int32),
            pltpu.VMEM((ep,), jnp.float32),
            pltpu.VMEM((256,), jnp.float32),
            pltpu.VMEM((_B, _H), jnp.float32),
            pltpu.VMEM((_B, _H), jnp.float32),
            pltpu.VMEM((_B, _H), jnp.float32),
            pltpu.VMEM((_B, _H), jnp.float32),
            pltpu.SemaphoreType.DMA,
            pltpu.SemaphoreType.DMA,
            pltpu.SemaphoreType.DMA,
            pltpu.SemaphoreType.DMA,
        ],
    )(a, b, src, dst)


def kernel(emb, edge_index, d, scale):
    src = edge_index[0].astype(jnp.int32)
    dst = edge_index[1].astype(jnp.int32)
    a, b = _prep(emb, d, scale)
    out = _edge_dot(a, b, src, dst)
    return out.reshape(-1, 1)
